# baseline (speedup 1.0000x reference)
import jax
import jax.numpy as jnp
from jax.experimental import pallas as pl


def _copy_body(x_ref, o_ref):
    o_ref[...] = x_ref[...]


def kernel(initial_space, finite_space, time_embedding):
    return pl.pallas_call(
        _copy_body,
        out_shape=jax.ShapeDtypeStruct(initial_space.shape, initial_space.dtype),
    )(initial_space)
